# baseline (speedup 1.0000x reference)
import functools

import jax
import jax.numpy as jnp
from jax import lax
from jax.experimental import pallas as pl
from jax.experimental.pallas import tpu as pltpu
from jax.experimental.pallas import tpu_sc as plsc

B, S, D = 1, 2048, 1024
E, K = 8, 2
F = 512
F_SH = 1024
TOT = S * K
TMC = 256
NT = TOT // TMC
G = NT + E - 1
TM = 256
T = S // TM

NW = 32
RPW = TOT // NW


def _top2(logits):
    iota = lax.broadcasted_iota(jnp.int32, logits.shape, 1)
    m1 = jnp.max(logits, axis=-1, keepdims=True)
    i1 = jnp.min(jnp.where(logits == m1, iota, E), axis=-1, keepdims=True)
    l2 = jnp.where(iota == i1, -jnp.inf, logits)
    m2 = jnp.max(l2, axis=-1, keepdims=True)
    i2 = jnp.min(jnp.where(l2 == m2, iota, E), axis=-1, keepdims=True)
    w1 = 1.0 / (1.0 + jnp.exp(m2 - m1))
    w2 = 1.0 - w1
    return i1, i2, w1, w2


def _router_body(x_ref, wr_ref, logits_ref, pos_ref, sched_ref):
    x = x_ref[...]
    logits = jnp.dot(x, wr_ref[...], preferred_element_type=jnp.float32)
    logits_ref[...] = logits
    i1, i2, w1, w2 = _top2(logits)

    lanes = lax.broadcasted_iota(jnp.int32, (S, E), 1)
    ind = ((lanes == i1) | (lanes == i2)).astype(jnp.float32)
    run = ind
    k = 1
    while k < S:
        shifted = jnp.concatenate(
            [jnp.zeros((k, E), jnp.float32), run[: S - k, :]], axis=0)
        run = run + shifted
        k *= 2
    rank = run - ind
    counts = run[S - 1:, :]

    def excl_lane_cumsum(v):
        r = v
        k = 1
        while k < E:
            r = r + jnp.concatenate(
                [jnp.zeros((1, k), v.dtype), r[:, : E - k]], axis=1)
            k *= 2
        return r - v

    off = excl_lane_cumsum(counts)

    def pick(tab, idx):
        return jnp.sum(jnp.where(lanes == idx, tab, 0.0), axis=-1,
                       keepdims=True)

    pos1 = pick(off + rank, i1)
    pos2 = pick(off + rank, i2)
    pos_ref[...] = jnp.concatenate([pos1, pos2], axis=1).astype(jnp.int32)

    cnt_i = counts.astype(jnp.int32)
    off_i = off.astype(jnp.int32)
    first = off_i // TMC
    last = jnp.where(cnt_i > 0, (off_i + cnt_i - 1) // TMC, -1)
    items = jnp.where(cnt_i > 0, last - first + 1, 0)
    start = excl_lane_cumsum(items)
    total = jnp.sum(items, axis=-1, keepdims=True)

    gl = lax.broadcasted_iota(jnp.int32, (G, E), 1)
    gg = lax.broadcasted_iota(jnp.int32, (G, E), 0)
    e_of_g = jnp.sum(jnp.where(start <= gg, 1, 0), axis=-1, keepdims=True) - 1

    def pick_g(tab):
        return jnp.sum(jnp.where(gl == e_of_g, tab, 0), axis=-1, keepdims=True)

    g_iota = lax.broadcasted_iota(jnp.int32, (G, 1), 0)
    valid = g_iota < total
    t_of_g = pick_g(first) + (g_iota - pick_g(start))
    rs = jnp.maximum(pick_g(off_i), t_of_g * TMC)
    re = jnp.minimum(pick_g(off_i) + pick_g(cnt_i), (t_of_g + 1) * TMC)
    t_of_g = jnp.where(valid, t_of_g, NT - 1)
    e_of_g = jnp.where(valid, jnp.clip(e_of_g, 0, E - 1), E - 1)
    rs = jnp.where(valid, rs, 0)
    re = jnp.where(valid, re, 0)
    sched = jnp.concatenate([t_of_g, e_of_g, rs, re], axis=1)
    sched_ref[...] = sched


def _stage_a(x, W_router):
    return pl.pallas_call(
        _router_body,
        out_shape=[
            jax.ShapeDtypeStruct((S, E), jnp.float32),
            jax.ShapeDtypeStruct((S, K), jnp.int32),
            jax.ShapeDtypeStruct((G, 4), jnp.int32),
        ],
    )(x, W_router)


def _make_gather():
    mesh = plsc.VectorSubcoreMesh(core_axis_name="c", subcore_axis_name="s")
    tpw = S // NW

    @functools.partial(
        pl.kernel,
        out_type=jax.ShapeDtypeStruct((TOT, D), jnp.float32),
        mesh=mesh,
        scratch_types=[
            pltpu.VMEM((tpw,), jnp.int32),
            pltpu.VMEM((tpw,), jnp.int32),
            pltpu.VMEM((tpw, D), jnp.float32),
            pltpu.SemaphoreType.DMA,
            pltpu.SemaphoreType.DMA,
        ],
    )
    def gather_k(pos0_hbm, pos1_hbm, x_hbm, xs_hbm, i0_v, i1_v, rows_v,
                 s0, s1):
        wid = lax.axis_index("s") * 2 + lax.axis_index("c")
        tok = wid * tpw
        pltpu.sync_copy(pos0_hbm.at[pl.ds(tok, tpw)], i0_v)
        pltpu.sync_copy(pos1_hbm.at[pl.ds(tok, tpw)], i1_v)
        pltpu.sync_copy(x_hbm.at[pl.ds(tok, tpw)], rows_v)
        cp0 = pltpu.async_copy(rows_v, xs_hbm.at[i0_v], s0)
        cp1 = pltpu.async_copy(rows_v, xs_hbm.at[i1_v], s1)
        cp0.wait()
        cp1.wait()

    return gather_k


def _group_body(sched_ref, xs_ref, wr_ref, wg_ref, wu_ref, wd_ref, out_ref):
    g = pl.program_id(0)
    e = sched_ref[g, 1]
    rs = sched_ref[g, 2]
    re = sched_ref[g, 3]

    @pl.when(re > rs)
    def _work():
        xs = xs_ref[...]
        logits = jnp.dot(xs, wr_ref[...], preferred_element_type=jnp.float32)
        lanes = lax.broadcasted_iota(jnp.int32, (TMC, E), 1)
        m1 = jnp.max(logits, axis=-1, keepdims=True)
        l2 = jnp.where(logits == m1, -jnp.inf, logits)
        m2 = jnp.max(l2, axis=-1, keepdims=True)
        le = jnp.sum(jnp.where(lanes == e, logits, 0.0), axis=-1,
                     keepdims=True)
        w = jnp.exp(le - m1) / (1.0 + jnp.exp(m2 - m1))

        gg = jnp.dot(xs, wg_ref[0], preferred_element_type=jnp.float32)
        uu = jnp.dot(xs, wu_ref[0], preferred_element_type=jnp.float32)
        h = jax.nn.silu(gg) * uu
        o = jnp.dot(h, wd_ref[0], preferred_element_type=jnp.float32)

        row = (sched_ref[g, 0] * TMC
               + lax.broadcasted_iota(jnp.int32, (TMC, 1), 0))
        m = (row >= rs) & (row < re)
        out_ref[...] = jnp.where(m, w * o, out_ref[...])


def _stage_c(sched, xs, W_router, Wg, Wu, Wd):
    grid_spec = pltpu.PrefetchScalarGridSpec(
        num_scalar_prefetch=1,
        grid=(G,),
        in_specs=[
            pl.BlockSpec((TMC, D), lambda g, s: (s[g, 0], 0)),
            pl.BlockSpec((D, E), lambda g, s: (0, 0)),
            pl.BlockSpec((1, D, F), lambda g, s: (s[g, 1], 0, 0)),
            pl.BlockSpec((1, D, F), lambda g, s: (s[g, 1], 0, 0)),
            pl.BlockSpec((1, F, D), lambda g, s: (s[g, 1], 0, 0)),
        ],
        out_specs=pl.BlockSpec((TMC, D), lambda g, s: (s[g, 0], 0)),
    )
    return pl.pallas_call(
        _group_body,
        grid_spec=grid_spec,
        out_shape=jax.ShapeDtypeStruct((TOT, D), jnp.float32),
        compiler_params=pltpu.CompilerParams(
            dimension_semantics=("arbitrary",)),
    )(sched, xs, W_router, Wg, Wu, Wd)


def _make_ungather():
    mesh = plsc.VectorSubcoreMesh(core_axis_name="c", subcore_axis_name="s")
    tpw = S // NW

    @functools.partial(
        pl.kernel,
        out_type=jax.ShapeDtypeStruct((K, S, D), jnp.float32),
        mesh=mesh,
        scratch_types=[
            pltpu.VMEM((tpw,), jnp.int32),
            pltpu.VMEM((tpw,), jnp.int32),
            pltpu.VMEM((32, D), jnp.float32),
            pltpu.VMEM((32, D), jnp.float32),
            pltpu.SemaphoreType.DMA,
            pltpu.SemaphoreType.DMA,
        ],
    )
    def ungather_k(pos0_hbm, pos1_hbm, outs_hbm, pair_hbm,
                   p0_v, p1_v, buf0, buf1, sem0, sem1):
        wid = lax.axis_index("s") * 2 + lax.axis_index("c")
        tok0 = wid * tpw
        pltpu.sync_copy(pos0_hbm.at[pl.ds(tok0, tpw)], p0_v)
        pltpu.sync_copy(pos1_hbm.at[pl.ds(tok0, tpw)], p1_v)
        for c in range(tpw // 32):
            base = tok0 + c * 32
            cp0 = pltpu.async_copy(
                outs_hbm.at[p0_v.at[pl.ds(c * 32, 32)]], buf0, sem0)
            cp1 = pltpu.async_copy(
                outs_hbm.at[p1_v.at[pl.ds(c * 32, 32)]], buf1, sem1)
            cp0.wait()
            pltpu.sync_copy(buf0, pair_hbm.at[0, pl.ds(base, 32)])
            cp1.wait()
            pltpu.sync_copy(buf1, pair_hbm.at[1, pl.ds(base, 32)])

    return ungather_k


def _shared_body(x_ref, wsg_ref, wsu_ref, wsd_ref, wgate_ref, out_ref):
    x = x_ref[...]
    gate = jax.nn.sigmoid(jnp.dot(x, wgate_ref[...],
                                  preferred_element_type=jnp.float32))
    g = jnp.dot(x, wsg_ref[...], preferred_element_type=jnp.float32)
    u = jnp.dot(x, wsu_ref[...], preferred_element_type=jnp.float32)
    h = jax.nn.silu(g) * u
    o = jnp.dot(h, wsd_ref[...], preferred_element_type=jnp.float32)
    out_ref[...] = gate * o


def _stage_e(x, Wsg, Wsu, Wsd, Wse_gate):
    return pl.pallas_call(
        _shared_body,
        grid=(T,),
        in_specs=[
            pl.BlockSpec((TM, D), lambda t: (t, 0)),
            pl.BlockSpec((D, F_SH), lambda t: (0, 0)),
            pl.BlockSpec((D, F_SH), lambda t: (0, 0)),
            pl.BlockSpec((F_SH, D), lambda t: (0, 0)),
            pl.BlockSpec((D, 1), lambda t: (0, 0)),
        ],
        out_specs=pl.BlockSpec((TM, D), lambda t: (t, 0)),
        out_shape=jax.ShapeDtypeStruct((S, D), jnp.float32),
        compiler_params=pltpu.CompilerParams(
            dimension_semantics=("arbitrary",)),
    )(x, Wsg, Wsu, Wsd, Wse_gate)


def _combine_body(sh_ref, pair_ref, out_ref):
    out_ref[...] = sh_ref[...] + pair_ref[0] + pair_ref[1]


def _stage_f(sh, pair):
    return pl.pallas_call(
        _combine_body,
        grid=(T,),
        in_specs=[
            pl.BlockSpec((TM, D), lambda t: (t, 0)),
            pl.BlockSpec((K, TM, D), lambda t: (0, t, 0)),
        ],
        out_specs=pl.BlockSpec((TM, D), lambda t: (t, 0)),
        out_shape=jax.ShapeDtypeStruct((S, D), jnp.float32),
        compiler_params=pltpu.CompilerParams(
            dimension_semantics=("arbitrary",)),
    )(sh, pair)


@jax.jit
def kernel(hidden_states, W_router, Wg, Wu, Wd, Wsg, Wsu, Wsd, Wse_gate):
    x = hidden_states.reshape(S, D)
    logits, pos, sched = _stage_a(x, W_router)
    xs = _make_gather()(pos[:, 0], pos[:, 1], x)
    sh = _stage_e(x, Wsg, Wsu, Wsd, Wse_gate)
    outs = _stage_c(sched, xs, W_router, Wg, Wu, Wd)
    pair = _make_ungather()(pos[:, 0], pos[:, 1], outs)
    out = _stage_f(sh, pair)
    return out.reshape(B, S, D), logits.reshape(B, S, E)


# speedup vs baseline: 1.0382x; 1.0382x over previous
import functools

import jax
import jax.numpy as jnp
from jax import lax
from jax.experimental import pallas as pl
from jax.experimental.pallas import tpu as pltpu
from jax.experimental.pallas import tpu_sc as plsc

B, S, D = 1, 2048, 1024
E, K = 8, 2
F = 512
F_SH = 1024
TOT = S * K
TMC = 256
NT = TOT // TMC
G = NT + E - 1
TM = 256
T = S // TM

NW = 32
RPW = TOT // NW


def _top2(logits):
    iota = lax.broadcasted_iota(jnp.int32, logits.shape, 1)
    m1 = jnp.max(logits, axis=-1, keepdims=True)
    i1 = jnp.min(jnp.where(logits == m1, iota, E), axis=-1, keepdims=True)
    l2 = jnp.where(iota == i1, -jnp.inf, logits)
    m2 = jnp.max(l2, axis=-1, keepdims=True)
    i2 = jnp.min(jnp.where(l2 == m2, iota, E), axis=-1, keepdims=True)
    w1 = 1.0 / (1.0 + jnp.exp(m2 - m1))
    w2 = 1.0 - w1
    return i1, i2, w1, w2


def _router_body(x_ref, wr_ref, logits_ref, pos_ref, sched_ref):
    x = x_ref[...]
    logits = jnp.dot(x, wr_ref[...], preferred_element_type=jnp.float32)
    logits_ref[...] = logits
    i1, i2, w1, w2 = _top2(logits)

    lanes = lax.broadcasted_iota(jnp.int32, (S, E), 1)
    ind = ((lanes == i1) | (lanes == i2)).astype(jnp.float32)
    run = ind
    k = 1
    while k < S:
        shifted = jnp.concatenate(
            [jnp.zeros((k, E), jnp.float32), run[: S - k, :]], axis=0)
        run = run + shifted
        k *= 2
    rank = run - ind
    counts = run[S - 1:, :]

    def excl_lane_cumsum(v):
        r = v
        k = 1
        while k < E:
            r = r + jnp.concatenate(
                [jnp.zeros((1, k), v.dtype), r[:, : E - k]], axis=1)
            k *= 2
        return r - v

    off = excl_lane_cumsum(counts)

    def pick(tab, idx):
        return jnp.sum(jnp.where(lanes == idx, tab, 0.0), axis=-1,
                       keepdims=True)

    pos1 = pick(off + rank, i1)
    pos2 = pick(off + rank, i2)
    pos_ref[...] = jnp.concatenate([pos1, pos2], axis=1).astype(jnp.int32)

    cnt_i = counts.astype(jnp.int32)
    off_i = off.astype(jnp.int32)
    first = off_i // TMC
    last = jnp.where(cnt_i > 0, (off_i + cnt_i - 1) // TMC, -1)
    items = jnp.where(cnt_i > 0, last - first + 1, 0)
    start = excl_lane_cumsum(items)
    total = jnp.sum(items, axis=-1, keepdims=True)

    gl = lax.broadcasted_iota(jnp.int32, (G, E), 1)
    gg = lax.broadcasted_iota(jnp.int32, (G, E), 0)
    e_of_g = jnp.sum(jnp.where(start <= gg, 1, 0), axis=-1, keepdims=True) - 1

    def pick_g(tab):
        return jnp.sum(jnp.where(gl == e_of_g, tab, 0), axis=-1, keepdims=True)

    g_iota = lax.broadcasted_iota(jnp.int32, (G, 1), 0)
    valid = g_iota < total
    t_of_g = pick_g(first) + (g_iota - pick_g(start))
    rs = jnp.maximum(pick_g(off_i), t_of_g * TMC)
    re = jnp.minimum(pick_g(off_i) + pick_g(cnt_i), (t_of_g + 1) * TMC)
    t_of_g = jnp.where(valid, t_of_g, NT - 1)
    e_of_g = jnp.where(valid, jnp.clip(e_of_g, 0, E - 1), E - 1)
    rs = jnp.where(valid, rs, 0)
    re = jnp.where(valid, re, 0)
    sched = jnp.concatenate([t_of_g, e_of_g, rs, re], axis=1)
    sched_ref[...] = sched


def _stage_a(x, W_router):
    return pl.pallas_call(
        _router_body,
        out_shape=[
            jax.ShapeDtypeStruct((S, E), jnp.float32),
            jax.ShapeDtypeStruct((S, K), jnp.int32),
            jax.ShapeDtypeStruct((G, 4), jnp.int32),
        ],
    )(x, W_router)


def _make_gather():
    mesh = plsc.VectorSubcoreMesh(core_axis_name="c", subcore_axis_name="s")
    tpw = S // NW

    @functools.partial(
        pl.kernel,
        out_type=jax.ShapeDtypeStruct((TOT, D), jnp.float32),
        mesh=mesh,
        scratch_types=[
            pltpu.VMEM((tpw,), jnp.int32),
            pltpu.VMEM((tpw,), jnp.int32),
            pltpu.VMEM((tpw, D), jnp.float32),
            pltpu.SemaphoreType.DMA,
            pltpu.SemaphoreType.DMA,
        ],
    )
    def gather_k(pos0_hbm, pos1_hbm, x_hbm, xs_hbm, i0_v, i1_v, rows_v,
                 s0, s1):
        wid = lax.axis_index("s") * 2 + lax.axis_index("c")
        tok = wid * tpw
        pltpu.sync_copy(pos0_hbm.at[pl.ds(tok, tpw)], i0_v)
        pltpu.sync_copy(pos1_hbm.at[pl.ds(tok, tpw)], i1_v)
        pltpu.sync_copy(x_hbm.at[pl.ds(tok, tpw)], rows_v)
        cp0 = pltpu.async_copy(rows_v, xs_hbm.at[i0_v], s0)
        cp1 = pltpu.async_copy(rows_v, xs_hbm.at[i1_v], s1)
        cp0.wait()
        cp1.wait()

    return gather_k


def _group_body(sched_ref, xs_ref, wr_ref, wg_ref, wu_ref, wd_ref, out_ref):
    g = pl.program_id(0)
    e = sched_ref[g, 1]
    rs = sched_ref[g, 2]
    re = sched_ref[g, 3]

    @pl.when(re > rs)
    def _work():
        xs = xs_ref[...]
        logits = jnp.dot(xs, wr_ref[...], preferred_element_type=jnp.float32)
        lanes = lax.broadcasted_iota(jnp.int32, (TMC, E), 1)
        m1 = jnp.max(logits, axis=-1, keepdims=True)
        l2 = jnp.where(logits == m1, -jnp.inf, logits)
        m2 = jnp.max(l2, axis=-1, keepdims=True)
        le = jnp.sum(jnp.where(lanes == e, logits, 0.0), axis=-1,
                     keepdims=True)
        w = jnp.exp(le - m1) / (1.0 + jnp.exp(m2 - m1))

        gg = jnp.dot(xs, wg_ref[0], preferred_element_type=jnp.float32)
        uu = jnp.dot(xs, wu_ref[0], preferred_element_type=jnp.float32)
        h = jax.nn.silu(gg) * uu
        o = jnp.dot(h, wd_ref[0], preferred_element_type=jnp.float32)

        row = (sched_ref[g, 0] * TMC
               + lax.broadcasted_iota(jnp.int32, (TMC, 1), 0))
        m = (row >= rs) & (row < re)
        out_ref[...] = jnp.where(m, w * o, out_ref[...])


def _stage_c(sched, xs, W_router, Wg, Wu, Wd):
    grid_spec = pltpu.PrefetchScalarGridSpec(
        num_scalar_prefetch=1,
        grid=(G,),
        in_specs=[
            pl.BlockSpec((TMC, D), lambda g, s: (s[g, 0], 0)),
            pl.BlockSpec((D, E), lambda g, s: (0, 0)),
            pl.BlockSpec((1, D, F), lambda g, s: (s[g, 1], 0, 0)),
            pl.BlockSpec((1, D, F), lambda g, s: (s[g, 1], 0, 0)),
            pl.BlockSpec((1, F, D), lambda g, s: (s[g, 1], 0, 0)),
        ],
        out_specs=pl.BlockSpec((TMC, D), lambda g, s: (s[g, 0], 0)),
    )
    return pl.pallas_call(
        _group_body,
        grid_spec=grid_spec,
        out_shape=jax.ShapeDtypeStruct((TOT, D), jnp.float32),
        compiler_params=pltpu.CompilerParams(
            dimension_semantics=("arbitrary",)),
    )(sched, xs, W_router, Wg, Wu, Wd)


def _make_ungather():
    mesh = plsc.VectorSubcoreMesh(core_axis_name="c", subcore_axis_name="s")
    tpw = S // NW

    @functools.partial(
        pl.kernel,
        out_type=jax.ShapeDtypeStruct((K, S, D), jnp.float32),
        mesh=mesh,
        scratch_types=[
            pltpu.VMEM((tpw,), jnp.int32),
            pltpu.VMEM((tpw,), jnp.int32),
            pltpu.VMEM((32, D), jnp.float32),
            pltpu.VMEM((32, D), jnp.float32),
            pltpu.SemaphoreType.DMA,
            pltpu.SemaphoreType.DMA,
        ],
    )
    def ungather_k(pos0_hbm, pos1_hbm, outs_hbm, pair_hbm,
                   p0_v, p1_v, buf0, buf1, sem0, sem1):
        wid = lax.axis_index("s") * 2 + lax.axis_index("c")
        tok0 = wid * tpw
        pltpu.sync_copy(pos0_hbm.at[pl.ds(tok0, tpw)], p0_v)
        pltpu.sync_copy(pos1_hbm.at[pl.ds(tok0, tpw)], p1_v)
        for c in range(tpw // 32):
            base = tok0 + c * 32
            cp0 = pltpu.async_copy(
                outs_hbm.at[p0_v.at[pl.ds(c * 32, 32)]], buf0, sem0)
            cp1 = pltpu.async_copy(
                outs_hbm.at[p1_v.at[pl.ds(c * 32, 32)]], buf1, sem1)
            cp0.wait()
            pltpu.sync_copy(buf0, pair_hbm.at[0, pl.ds(base, 32)])
            cp1.wait()
            pltpu.sync_copy(buf1, pair_hbm.at[1, pl.ds(base, 32)])

    return ungather_k


def _shared_body(x_ref, wsg_ref, wsu_ref, wsd_ref, wgate_ref,
                 pair_ref, out_ref):
    x = x_ref[...]
    gate = jax.nn.sigmoid(jnp.dot(x, wgate_ref[...],
                                  preferred_element_type=jnp.float32))
    g = jnp.dot(x, wsg_ref[...], preferred_element_type=jnp.float32)
    u = jnp.dot(x, wsu_ref[...], preferred_element_type=jnp.float32)
    h = jax.nn.silu(g) * u
    o = jnp.dot(h, wsd_ref[...], preferred_element_type=jnp.float32)
    out_ref[...] = gate * o + pair_ref[0] + pair_ref[1]


def _stage_e(x, Wsg, Wsu, Wsd, Wse_gate, pair):
    return pl.pallas_call(
        _shared_body,
        grid=(T,),
        in_specs=[
            pl.BlockSpec((TM, D), lambda t: (t, 0)),
            pl.BlockSpec((D, F_SH), lambda t: (0, 0)),
            pl.BlockSpec((D, F_SH), lambda t: (0, 0)),
            pl.BlockSpec((F_SH, D), lambda t: (0, 0)),
            pl.BlockSpec((D, 1), lambda t: (0, 0)),
            pl.BlockSpec((K, TM, D), lambda t: (0, t, 0)),
        ],
        out_specs=pl.BlockSpec((TM, D), lambda t: (t, 0)),
        out_shape=jax.ShapeDtypeStruct((S, D), jnp.float32),
        compiler_params=pltpu.CompilerParams(
            dimension_semantics=("arbitrary",)),
    )(x, Wsg, Wsu, Wsd, Wse_gate, pair)


@jax.jit
def kernel(hidden_states, W_router, Wg, Wu, Wd, Wsg, Wsu, Wsd, Wse_gate):
    x = hidden_states.reshape(S, D)
    logits, pos, sched = _stage_a(x, W_router)
    xs = _make_gather()(pos[:, 0], pos[:, 1], x)
    outs = _stage_c(sched, xs, W_router, Wg, Wu, Wd)
    pair = _make_ungather()(pos[:, 0], pos[:, 1], outs)
    out = _stage_e(x, Wsg, Wsu, Wsd, Wse_gate, pair)
    return out.reshape(B, S, D), logits.reshape(B, S, E)
